# baseline (speedup 1.0000x reference)
import functools

import jax
import jax.numpy as jnp
from jax.experimental import pallas as pl
from jax.experimental.pallas import tpu as pltpu

P = 1048576
CIN = 32
COUT = 32
S = 4


_CHUNK = 16384


def _body(ld_ref, lev_ref, x_ref, w_ref, b_ref, o_ref):
    delta = ld_ref[0]
    block = x_ref.shape[2]
    chunk = min(_CHUNK, block)
    for c in range(0, block, chunk):
        sl = pl.ds(c, chunk)
        s = jnp.clip(lev_ref[:, sl] + delta, 0, S - 1)
        xb = x_ref[0, :, sl]
        y = jnp.dot(w_ref[:], xb, preferred_element_type=jnp.float32)
        ya = jnp.where(s == 0, y[0:COUT, :], y[COUT:2 * COUT, :])
        yb = jnp.where(s == 2, y[2 * COUT:3 * COUT, :], y[3 * COUT:, :])
        o_ref[0, :, sl] = jnp.where(s <= 1, ya, yb) + b_ref[:]


@functools.partial(jax.jit, static_argnames=("block",))
def _run(x, levels2d, level_deltas, wstack, bias2d, block=2048):
    p = x.shape[2]
    block = min(block, p)
    grid = (p // block,)
    return pl.pallas_call(
        _body,
        grid=grid,
        in_specs=[
            pl.BlockSpec(memory_space=pltpu.SMEM),
            pl.BlockSpec((1, block), lambda i: (0, i)),
            pl.BlockSpec((1, CIN, block), lambda i: (0, 0, i)),
            pl.BlockSpec((S * COUT, CIN), lambda i: (0, 0)),
            pl.BlockSpec((COUT, 1), lambda i: (0, 0)),
        ],
        out_specs=pl.BlockSpec((1, COUT, block), lambda i: (0, 0, i)),
        out_shape=jax.ShapeDtypeStruct((1, COUT, p), x.dtype),
        compiler_params=pltpu.CompilerParams(
            dimension_semantics=("parallel",),
        ),
    )(level_deltas, levels2d, x, wstack, bias2d)


def kernel(input_features, levels, level_deltas, weight, bias):
    wstack = weight.reshape(S * COUT, CIN)
    levels2d = levels.reshape(1, -1)
    bias2d = bias.reshape(COUT, 1)
    return _run(input_features, levels2d, level_deltas, wstack, bias2d,
                block=65536)
